# baseline (speedup 1.0000x reference)
import functools

import jax
import jax.numpy as jnp
from jax import lax
from jax.experimental import pallas as pl
from jax.experimental.pallas import tpu as pltpu
from jax.experimental.pallas import tpu_sc as plsc

N_EMBD = 64
NC = 2
NS = 16
NW = NC * NS

BATCH = 4096
HIST = 200
TOTAL = BATCH * HIST
B_PER_W = TOTAL // NW
CHUNK = 512
N_CHUNKS = B_PER_W // CHUNK

_mesh = plsc.VectorSubcoreMesh(core_axis_name="c", subcore_axis_name="s")


@functools.partial(
    pl.kernel,
    mesh=_mesh,
    out_type=jax.ShapeDtypeStruct((TOTAL, N_EMBD), jnp.float32),
    scratch_types=[
        pltpu.VMEM((CHUNK,), jnp.int32),
        pltpu.VMEM((CHUNK, N_EMBD), jnp.float32),
        pltpu.SemaphoreType.DMA,
    ],
    compiler_params=pltpu.CompilerParams(use_tc_tiling_on_sc=False),
)
def _gather_rows(table_hbm, idx_hbm, out_hbm, idx_v, rows_v, sem):
    wid = lax.axis_index("s") * NC + lax.axis_index("c")
    base = wid * B_PER_W

    def body(i, carry):
        off = base + i * CHUNK
        pltpu.sync_copy(idx_hbm.at[pl.ds(off, CHUNK)], idx_v)
        pltpu.async_copy(table_hbm.at[idx_v], rows_v, sem).wait()
        pltpu.sync_copy(rows_v, out_hbm.at[pl.ds(off, CHUNK)])
        return carry

    lax.fori_loop(0, N_CHUNKS, body, 0)


def kernel(x, food_vectors):
    flat = x.reshape(TOTAL)
    out = _gather_rows(food_vectors, flat)
    return out.reshape(BATCH, HIST, N_EMBD)


# speedup vs baseline: 1.0464x; 1.0464x over previous
import functools

import jax
import jax.numpy as jnp
from jax import lax
from jax.experimental import pallas as pl
from jax.experimental.pallas import tpu as pltpu
from jax.experimental.pallas import tpu_sc as plsc

N_EMBD = 64
NC = 2
NS = 16
NW = NC * NS

BATCH = 4096
HIST = 200
TOTAL = BATCH * HIST
B_PER_W = TOTAL // NW
CHUNK = 512
N_CHUNKS = B_PER_W // CHUNK

_mesh = plsc.VectorSubcoreMesh(core_axis_name="c", subcore_axis_name="s")


@functools.partial(
    pl.kernel,
    mesh=_mesh,
    out_type=jax.ShapeDtypeStruct((TOTAL, N_EMBD), jnp.float32),
    scratch_types=[
        pltpu.VMEM((B_PER_W,), jnp.int32),
        pltpu.VMEM((CHUNK, N_EMBD), jnp.float32),
        pltpu.VMEM((CHUNK, N_EMBD), jnp.float32),
        pltpu.SemaphoreType.DMA,
        pltpu.SemaphoreType.DMA,
        pltpu.SemaphoreType.DMA,
        pltpu.SemaphoreType.DMA,
    ],
    compiler_params=pltpu.CompilerParams(use_tc_tiling_on_sc=False),
)
def _gather_rows(table_hbm, idx_hbm, out_hbm, idx_v, rows0, rows1,
                 gsem0, gsem1, ssem0, ssem1):
    wid = lax.axis_index("s") * NC + lax.axis_index("c")
    base = wid * B_PER_W

    pltpu.sync_copy(idx_hbm.at[pl.ds(base, B_PER_W)], idx_v)

    def gather_start(i, buf, sem):
        return pltpu.async_copy(
            table_hbm.at[idx_v.at[pl.ds(i * CHUNK, CHUNK)]], buf, sem)

    def scatter_start(i, buf, sem):
        return pltpu.async_copy(
            buf, out_hbm.at[pl.ds(base + i * CHUNK, CHUNK)], sem)

    gather_start(0, rows0, gsem0)

    def body(i, carry):
        even = i % 2 == 0

        def step(rows_cur, rows_nxt, gsem_cur, gsem_nxt, ssem_cur, ssem_nxt):
            @pl.when(i + 1 < N_CHUNKS)
            def _():
                @pl.when(i >= 1)
                def _():
                    pltpu.make_async_copy(
                        rows_nxt, out_hbm.at[pl.ds(0, CHUNK)], ssem_nxt).wait()
                gather_start(i + 1, rows_nxt, gsem_nxt)

            pltpu.make_async_copy(
                table_hbm.at[idx_v.at[pl.ds(0, CHUNK)]], rows_cur, gsem_cur
            ).wait()
            scatter_start(i, rows_cur, ssem_cur)

        @pl.when(even)
        def _():
            step(rows0, rows1, gsem0, gsem1, ssem0, ssem1)

        @pl.when(jnp.logical_not(even))
        def _():
            step(rows1, rows0, gsem1, gsem0, ssem1, ssem0)

        return carry

    lax.fori_loop(0, N_CHUNKS, body, 0)

    pltpu.make_async_copy(rows0, out_hbm.at[pl.ds(0, CHUNK)], ssem0).wait()
    pltpu.make_async_copy(rows1, out_hbm.at[pl.ds(0, CHUNK)], ssem1).wait()


def kernel(x, food_vectors):
    flat = x.reshape(TOTAL)
    out = _gather_rows(food_vectors, flat)
    return out.reshape(BATCH, HIST, N_EMBD)


# speedup vs baseline: 1.3899x; 1.3283x over previous
import functools

import jax
import jax.numpy as jnp
from jax import lax
from jax.experimental import pallas as pl
from jax.experimental.pallas import tpu as pltpu
from jax.experimental.pallas import tpu_sc as plsc

N_EMBD = 64
PAD = 128
NC = 2
NS = 16
NW = NC * NS

BATCH = 4096
HIST = 200
TOTAL = BATCH * HIST
B_PER_W = TOTAL // NW
ROWS_PER_W = BATCH // NW
KB = 2
CHUNK = KB * HIST
N_CHUNKS = ROWS_PER_W // KB

_mesh = plsc.VectorSubcoreMesh(core_axis_name="c", subcore_axis_name="s")


@functools.partial(
    pl.kernel,
    mesh=_mesh,
    out_type=jax.ShapeDtypeStruct((BATCH, HIST, PAD), jnp.float32),
    scratch_types=[
        pltpu.VMEM((B_PER_W,), jnp.int32),
        pltpu.VMEM((CHUNK, N_EMBD), jnp.float32),
        pltpu.VMEM((CHUNK, N_EMBD), jnp.float32),
        pltpu.SemaphoreType.DMA,
        pltpu.SemaphoreType.DMA,
        pltpu.SemaphoreType.DMA,
        pltpu.SemaphoreType.DMA,
    ],
    compiler_params=pltpu.CompilerParams(use_tc_tiling_on_sc=False),
)
def _gather_rows(table_hbm, idx_hbm, out_hbm, idx_v, rows0, rows1,
                 gsem0, gsem1, ssem0, ssem1):
    wid = lax.axis_index("s") * NC + lax.axis_index("c")
    base = wid * B_PER_W
    row_base = wid * ROWS_PER_W

    pltpu.sync_copy(idx_hbm.at[pl.ds(base, B_PER_W)], idx_v)

    def gather_start(i, buf, sem):
        return pltpu.async_copy(
            table_hbm.at[idx_v.at[pl.ds(i * CHUNK, CHUNK)]], buf, sem)

    def scatter_start(i, buf, sem):
        b0 = row_base + i * KB
        for j in range(KB):
            pltpu.async_copy(
                buf.at[pl.ds(j * HIST, HIST)],
                out_hbm.at[b0 + j, :, pl.ds(0, N_EMBD)],
                sem)

    def scatter_wait(buf, sem):
        for j in range(KB):
            pltpu.make_async_copy(
                buf.at[pl.ds(j * HIST, HIST)],
                out_hbm.at[0, :, pl.ds(0, N_EMBD)],
                sem).wait()

    gather_start(0, rows0, gsem0)

    def body(i, carry):
        even = i % 2 == 0

        def step(rows_cur, rows_nxt, gsem_cur, gsem_nxt, ssem_cur, ssem_nxt):
            @pl.when(i + 1 < N_CHUNKS)
            def _():
                @pl.when(i >= 1)
                def _():
                    scatter_wait(rows_nxt, ssem_nxt)
                gather_start(i + 1, rows_nxt, gsem_nxt)

            pltpu.make_async_copy(
                table_hbm.at[idx_v.at[pl.ds(0, CHUNK)]], rows_cur, gsem_cur
            ).wait()
            scatter_start(i, rows_cur, ssem_cur)

        @pl.when(even)
        def _():
            step(rows0, rows1, gsem0, gsem1, ssem0, ssem1)

        @pl.when(jnp.logical_not(even))
        def _():
            step(rows1, rows0, gsem1, gsem0, ssem1, ssem0)

        return carry

    lax.fori_loop(0, N_CHUNKS, body, 0)

    scatter_wait(rows0, ssem0)
    scatter_wait(rows1, ssem1)


def kernel(x, food_vectors):
    flat = x.reshape(TOTAL)
    out = _gather_rows(food_vectors, flat)
    return out[:, :, :N_EMBD]


# speedup vs baseline: 1.3913x; 1.0010x over previous
import functools

import jax
import jax.numpy as jnp
from jax import lax
from jax.experimental import pallas as pl
from jax.experimental.pallas import tpu as pltpu
from jax.experimental.pallas import tpu_sc as plsc

N_EMBD = 64
PAD = 128
NC = 2
NS = 16
NW = NC * NS

BATCH = 4096
HIST = 200
TOTAL = BATCH * HIST
B_PER_W = TOTAL // NW
ROWS_PER_W = BATCH // NW
KB = 2
CHUNK = KB * HIST
N_CHUNKS = ROWS_PER_W // KB

_mesh = plsc.VectorSubcoreMesh(core_axis_name="c", subcore_axis_name="s")


@functools.partial(
    pl.kernel,
    mesh=_mesh,
    out_type=jax.ShapeDtypeStruct((BATCH, HIST, PAD), jnp.float32),
    scratch_types=[
        pltpu.VMEM((B_PER_W,), jnp.int32),
        pltpu.VMEM((CHUNK, N_EMBD), jnp.float32),
        pltpu.VMEM((CHUNK, N_EMBD), jnp.float32),
        pltpu.SemaphoreType.DMA,
        pltpu.SemaphoreType.DMA,
        pltpu.SemaphoreType.DMA,
        pltpu.SemaphoreType.DMA,
    ],
    compiler_params=pltpu.CompilerParams(use_tc_tiling_on_sc=False),
)
def _gather_rows(table_hbm, idx_hbm, out_hbm, idx_v, rows0, rows1,
                 gsem0, gsem1, ssem0, ssem1):
    wid = lax.axis_index("s") * NC + lax.axis_index("c")
    base = wid * B_PER_W
    row_base = wid * ROWS_PER_W

    pltpu.sync_copy(idx_hbm.at[pl.ds(base, B_PER_W)], idx_v)

    def gather_start(i, buf, sem):
        return pltpu.async_copy(
            table_hbm.at[idx_v.at[pl.ds(i * CHUNK, CHUNK)]], buf, sem)

    def scatter_start(i, buf, sem):
        b0 = row_base + i * KB
        for j in range(KB):
            pltpu.async_copy(
                buf.at[pl.ds(j * HIST, HIST)],
                out_hbm.at[b0 + j, :, pl.ds(0, N_EMBD)],
                sem)

    def scatter_wait(buf, sem):
        for j in range(KB):
            pltpu.make_async_copy(
                buf.at[pl.ds(j * HIST, HIST)],
                out_hbm.at[0, :, pl.ds(0, N_EMBD)],
                sem).wait()

    gather_start(0, rows0, gsem0)

    def body(i, carry):
        even = i % 2 == 0

        def step(rows_cur, rows_nxt, gsem_cur, gsem_nxt, ssem_cur, ssem_nxt):
            @pl.when(i + 1 < N_CHUNKS)
            def _():
                @pl.when(i >= 1)
                def _():
                    scatter_wait(rows_nxt, ssem_nxt)
                gather_start(i + 1, rows_nxt, gsem_nxt)

            pltpu.make_async_copy(
                table_hbm.at[idx_v.at[pl.ds(0, CHUNK)]], rows_cur, gsem_cur
            ).wait()
            scatter_start(i, rows_cur, ssem_cur)

        @pl.when(even)
        def _():
            step(rows0, rows1, gsem0, gsem1, ssem0, ssem1)

        @pl.when(jnp.logical_not(even))
        def _():
            step(rows1, rows0, gsem1, gsem0, ssem1, ssem0)

        return carry

    lax.fori_loop(0, N_CHUNKS, body, 0)

    scatter_wait(rows0, ssem0)
    scatter_wait(rows1, ssem1)


def kernel(x, food_vectors):
    flat = x.reshape(TOTAL)
    flat_tab = jax.lax.optimization_barrier(food_vectors.reshape(-1))
    table = flat_tab.reshape(food_vectors.shape)
    out = _gather_rows(table, flat)
    return out[:, :, :N_EMBD]
